# baseline (speedup 1.0000x reference)
import functools

import jax
import jax.numpy as jnp
from jax import lax
from jax.experimental import pallas as pl
from jax.experimental.pallas import tpu as pltpu
from jax.experimental.pallas import tpu_sc as plsc

N = 10000
D = 128
T = 6
L = 2
M = 100000
E = T * M
SMALL = 1e-8
F32 = jnp.float32

NC, NS = 2, 16
NW = NC * NS
CH = 128
NBUF = 3
C = NBUF * (-(-E // (NW * CH * NBUF)))
CA = C + NBUF
E_PAD = NW * CA * CH
TRASH = N
SH = 10112
ZR = SH // NS
ZB = 8

BN = 1000
NB = N // BN


def _zero_shared(zrows, shared, sid):
    def zstep(i, carry):
        pltpu.sync_copy(zrows, shared.at[pl.ds(sid * ZR + i * ZB, ZB)])
        return carry
    lax.fori_loop(0, ZR // ZB, zstep, 0)


def _sc_msgs_body(prop, idx, zrows, out,
                  i0, i1, i2, r0, r1, r2,
                  si0, si1, si2, g0, g1, g2, s0, s1, s2, shared):
    cid = lax.axis_index("c")
    sid = lax.axis_index("s")
    w = sid * NC + cid
    idxb = (i0, i1, i2)
    rows = (r0, r1, r2)
    si = (si0, si1, si2)
    sg = (g0, g1, g2)
    ss = (s0, s1, s2)
    _zero_shared(zrows, shared, sid)
    plsc.subcore_barrier()
    for b in range(NBUF):
        pltpu.async_copy(idx.at[w, b], idxb[b], si[b])
    for b in range(NBUF):
        pltpu.make_async_copy(idx.at[w, b], idxb[b], si[b]).wait()
        pltpu.async_copy(prop.at[idxb[b].at[0]], rows[b], sg[b])

    def step(j, carry):
        c = NBUF * j
        for b in range(NBUF):
            pltpu.make_async_copy(prop.at[idxb[b].at[0]],
                                  rows[b], sg[b]).wait()
            pltpu.async_copy(rows[b], shared.at[idxb[b].at[1]],
                             ss[b], add=True)
        for b in range(NBUF):
            pltpu.make_async_copy(rows[b], shared.at[idxb[b].at[1]],
                                  ss[b]).wait()
            pltpu.async_copy(idx.at[w, c + b + NBUF], idxb[b], si[b])
        for b in range(NBUF):
            pltpu.make_async_copy(idx.at[w, c + b + NBUF],
                                  idxb[b], si[b]).wait()
            pltpu.async_copy(prop.at[idxb[b].at[0]], rows[b], sg[b])
        return carry

    lax.fori_loop(0, C // NBUF, step, 0)
    for b in range(NBUF):
        pltpu.make_async_copy(prop.at[idxb[b].at[0]], rows[b], sg[b]).wait()
    plsc.subcore_barrier()

    @pl.when(sid < 10)
    def _():
        pltpu.sync_copy(shared.at[pl.ds(sid * 1000, 1000)],
                        out.at[cid, pl.ds(sid * 1000, 1000)])


@functools.cache
def _sc_kernels():
    mesh = plsc.VectorSubcoreMesh(core_axis_name="c", subcore_axis_name="s",
                                  num_cores=NC, num_subcores=NS)
    sc_msgs = pl.kernel(
        _sc_msgs_body,
        out_type=jax.ShapeDtypeStruct((NC, N, D), F32),
        mesh=mesh,
        scratch_types=(
            [pltpu.VMEM((2, CH), jnp.int32)] * NBUF
            + [pltpu.VMEM((CH, D), F32)] * NBUF
            + [pltpu.SemaphoreType.DMA] * (3 * NBUF)
            + [pltpu.VMEM_SHARED((SH, D), F32)]
        ),
    )
    sc_binc = pl.kernel(
        _sc_binc_body,
        out_type=jax.ShapeDtypeStruct((NC, N, D), F32),
        mesh=mesh,
        scratch_types=(
            [pltpu.VMEM((2, CH), jnp.int32)] * NBUF
            + [pltpu.SemaphoreType.DMA] * NBUF
            + [pltpu.VMEM((CH, D), F32)]
            + [pltpu.VMEM_SHARED((SH, D), F32)]
        ),
    )
    return sc_msgs, sc_binc


def _sc_binc_body(idx, zrows, ones_h, out,
                  i0, i1, i2, si0, si1, si2, ones_v, shared):
    cid = lax.axis_index("c")
    sid = lax.axis_index("s")
    w = sid * NC + cid
    idxb = (i0, i1, i2)
    si = (si0, si1, si2)
    for b in range(NBUF):
        pltpu.async_copy(idx.at[w, b], idxb[b], si[b])
    _zero_shared(zrows, shared, sid)
    pltpu.sync_copy(ones_h, ones_v)
    plsc.subcore_barrier()

    def step(j, carry):
        c = NBUF * j
        for b in range(NBUF):
            pltpu.make_async_copy(idx.at[w, c + b], idxb[b], si[b]).wait()
            pltpu.sync_copy(ones_v, shared.at[idxb[b].at[1]], add=True)
            pltpu.async_copy(idx.at[w, c + b + NBUF], idxb[b], si[b])
        return carry

    lax.fori_loop(0, C // NBUF, step, 0)
    for b in range(NBUF):
        pltpu.make_async_copy(idx.at[w, C + b], idxb[b], si[b]).wait()
    plsc.subcore_barrier()

    @pl.when(sid < 10)
    def _():
        pltpu.sync_copy(shared.at[pl.ds(sid * 1000, 1000)],
                        out.at[cid, pl.ds(sid * 1000, 1000)])


def _prop_body(h_ref, w_ref, b_ref, o_ref):
    o_ref[0] = (jnp.dot(h_ref[...], w_ref[0], preferred_element_type=F32)
                + b_ref[0])


def _tc_prop(h, wt, bt):
    return pl.pallas_call(
        _prop_body,
        grid=(NB, T),
        in_specs=[
            pl.BlockSpec((BN, D), lambda nb, t: (nb, 0)),
            pl.BlockSpec((1, D, D), lambda nb, t: (t, 0, 0)),
            pl.BlockSpec((1, 1, D), lambda nb, t: (t, 0, 0)),
        ],
        out_specs=pl.BlockSpec((1, BN, D), lambda nb, t: (t, nb, 0)),
        out_shape=jax.ShapeDtypeStruct((T, N, D), F32),
    )(h, wt, bt)


def _gru_body(p_ref, bc_ref, h_ref, wih_ref, whh_ref, bih_ref, bhh_ref, o_ref):
    cnt = bc_ref[0, :, 0] + bc_ref[1, :, 0]
    div = jnp.where(cnt == 0.0, 1.0, cnt)
    msgs = (p_ref[0] + p_ref[1]) / div[:, None] + SMALL
    h = h_ref[...]
    gi = jnp.dot(msgs, wih_ref[...], preferred_element_type=F32) + bih_ref[...]
    gh = jnp.dot(h, whh_ref[...], preferred_element_type=F32) + bhh_ref[...]
    r = jax.nn.sigmoid(gi[:, :D] + gh[:, :D])
    z = jax.nn.sigmoid(gi[:, D:2 * D] + gh[:, D:2 * D])
    n = jnp.tanh(gi[:, 2 * D:] + r * gh[:, 2 * D:])
    o_ref[...] = (1.0 - z) * n + z * h


def _tc_gru(part, bc, h, wih_t, whh_t, bih, bhh):
    return pl.pallas_call(
        _gru_body,
        grid=(NB,),
        in_specs=[
            pl.BlockSpec((NC, BN, D), lambda nb: (0, nb, 0)),
            pl.BlockSpec((NC, BN, D), lambda nb: (0, nb, 0)),
            pl.BlockSpec((BN, D), lambda nb: (nb, 0)),
            pl.BlockSpec((D, 3 * D), lambda nb: (0, 0)),
            pl.BlockSpec((D, 3 * D), lambda nb: (0, 0)),
            pl.BlockSpec((1, 3 * D), lambda nb: (0, 0)),
            pl.BlockSpec((1, 3 * D), lambda nb: (0, 0)),
        ],
        out_specs=pl.BlockSpec((BN, D), lambda nb: (nb, 0)),
        out_shape=jax.ShapeDtypeStruct((N, D), F32),
    )(part, bc, h, wih_t, whh_t, bih, bhh)


def kernel(edge_lists, node_states, W_msg, b_msg, W_ih, W_hh, b_ih, b_hh):
    h = node_states

    gidx = (edge_lists[:, :, 0]
            + (jnp.arange(T, dtype=jnp.int32) * N)[:, None]).reshape(-1)
    tgt = edge_lists[:, :, 1].reshape(-1)
    pad = NW * C * CH - E
    gidx = jnp.concatenate([gidx, jnp.zeros((pad,), jnp.int32)])
    tgt = jnp.concatenate([tgt, jnp.full((pad,), TRASH, jnp.int32)])
    g3 = jnp.concatenate([gidx.reshape(NW, C, CH),
                          jnp.zeros((NW, NBUF, CH), jnp.int32)], axis=1)
    t3 = jnp.concatenate([tgt.reshape(NW, C, CH),
                          jnp.full((NW, NBUF, CH), TRASH, jnp.int32)], axis=1)
    idx_all = jnp.stack([g3, t3], axis=2)

    zrows = jnp.zeros((ZB, D), F32)
    ones_rows = jnp.ones((CH, D), F32)

    sc_msgs, sc_binc = _sc_kernels()
    bc = sc_binc(idx_all, zrows, ones_rows)

    for layer in range(L):
        wt = W_msg[layer].reshape(T, D, D).transpose(0, 2, 1)
        bt = b_msg[layer].reshape(T, 1, D)
        prop = _tc_prop(h, wt, bt)
        part = sc_msgs(prop.reshape(T * N, D), idx_all, zrows)
        h = _tc_gru(part, bc, h,
                    W_ih[layer].T, W_hh[layer].T,
                    b_ih[layer].reshape(1, 3 * D),
                    b_hh[layer].reshape(1, 3 * D))
    return h


# speedup vs baseline: 1.6415x; 1.6415x over previous
import functools

import jax
import jax.numpy as jnp
from jax import lax
from jax.experimental import pallas as pl
from jax.experimental.pallas import tpu as pltpu
from jax.experimental.pallas import tpu_sc as plsc

N = 10000
D = 128
T = 6
L = 2
M = 100000
E = T * M
SMALL = 1e-8
F32 = jnp.float32

NC, NS = 2, 16
NW = NC * NS
CH = 128
NBUF = 3
C = NBUF * (-(-E // (NW * CH * NBUF)))
CA = C
E_PAD = NW * CA * CH
TRASH = N
SH = 10112
ZR = SH // NS
ZB = 8

BN = 1000
NB = N // BN


def _zero_shared(zrows, shared, sid):
    def zstep(i, carry):
        pltpu.sync_copy(zrows, shared.at[pl.ds(sid * ZR + i * ZB, ZB)])
        return carry
    lax.fori_loop(0, ZR // ZB, zstep, 0)


def _sc_msgs_body(prop, idx, zrows, out,
                  idxb, r0, r1, r2, g0, g1, g2, shared):
    cid = lax.axis_index("c")
    sid = lax.axis_index("s")
    w = sid * NC + cid
    rows = (r0, r1, r2)
    sg = (g0, g1, g2)
    _zero_shared(zrows, shared, sid)
    plsc.subcore_barrier()

    def step(j, carry):
        c = NBUF * j
        pltpu.sync_copy(idx.at[w, pl.ds(c, NBUF)], idxb)
        cps = [pltpu.async_copy(prop.at[idxb.at[b, 0]], rows[b], sg[b])
               for b in range(NBUF)]
        for b in range(NBUF):
            cps[b].wait()
            pltpu.sync_copy(rows[b], shared.at[idxb.at[b, 1]], add=True)
        return carry

    lax.fori_loop(0, C // NBUF, step, 0)
    plsc.subcore_barrier()

    @pl.when(sid < 10)
    def _():
        pltpu.sync_copy(shared.at[pl.ds(sid * 1000, 1000)],
                        out.at[cid, pl.ds(sid * 1000, 1000)])


@functools.cache
def _sc_kernels():
    mesh = plsc.VectorSubcoreMesh(core_axis_name="c", subcore_axis_name="s",
                                  num_cores=NC, num_subcores=NS)
    sc_msgs = pl.kernel(
        _sc_msgs_body,
        out_type=jax.ShapeDtypeStruct((NC, N, D), F32),
        mesh=mesh,
        scratch_types=(
            [pltpu.VMEM((NBUF, 2, CH), jnp.int32)]
            + [pltpu.VMEM((CH, D), F32)] * NBUF
            + [pltpu.SemaphoreType.DMA] * NBUF
            + [pltpu.VMEM_SHARED((SH, D), F32)]
        ),
    )
    sc_binc = pl.kernel(
        _sc_binc_body,
        out_type=jax.ShapeDtypeStruct((NC, N, D), F32),
        mesh=mesh,
        scratch_types=[
            pltpu.VMEM((NBUF, 2, CH), jnp.int32),
            pltpu.VMEM((CH, D), F32),
            pltpu.VMEM_SHARED((SH, D), F32),
        ],
    )
    return sc_msgs, sc_binc


def _sc_binc_body(idx, zrows, ones_h, out, idxb, ones_v, shared):
    cid = lax.axis_index("c")
    sid = lax.axis_index("s")
    w = sid * NC + cid
    _zero_shared(zrows, shared, sid)
    pltpu.sync_copy(ones_h, ones_v)
    plsc.subcore_barrier()

    def step(j, carry):
        c = NBUF * j
        pltpu.sync_copy(idx.at[w, pl.ds(c, NBUF)], idxb)
        for b in range(NBUF):
            pltpu.sync_copy(ones_v, shared.at[idxb.at[b, 1]], add=True)
        return carry

    lax.fori_loop(0, C // NBUF, step, 0)
    plsc.subcore_barrier()

    @pl.when(sid < 10)
    def _():
        pltpu.sync_copy(shared.at[pl.ds(sid * 1000, 1000)],
                        out.at[cid, pl.ds(sid * 1000, 1000)])


def _prop_body(h_ref, w_ref, b_ref, o_ref):
    o_ref[0] = (jnp.dot(h_ref[...], w_ref[0], preferred_element_type=F32)
                + b_ref[0])


def _tc_prop(h, wt, bt):
    return pl.pallas_call(
        _prop_body,
        grid=(NB, T),
        in_specs=[
            pl.BlockSpec((BN, D), lambda nb, t: (nb, 0)),
            pl.BlockSpec((1, D, D), lambda nb, t: (t, 0, 0)),
            pl.BlockSpec((1, 1, D), lambda nb, t: (t, 0, 0)),
        ],
        out_specs=pl.BlockSpec((1, BN, D), lambda nb, t: (t, nb, 0)),
        out_shape=jax.ShapeDtypeStruct((T, N, D), F32),
    )(h, wt, bt)


def _gru_body(p_ref, bc_ref, h_ref, wih_ref, whh_ref, bih_ref, bhh_ref, o_ref):
    cnt = bc_ref[0, :, 0] + bc_ref[1, :, 0]
    div = jnp.where(cnt == 0.0, 1.0, cnt)
    msgs = (p_ref[0] + p_ref[1]) / div[:, None] + SMALL
    h = h_ref[...]
    gi = jnp.dot(msgs, wih_ref[...], preferred_element_type=F32) + bih_ref[...]
    gh = jnp.dot(h, whh_ref[...], preferred_element_type=F32) + bhh_ref[...]
    r = jax.nn.sigmoid(gi[:, :D] + gh[:, :D])
    z = jax.nn.sigmoid(gi[:, D:2 * D] + gh[:, D:2 * D])
    n = jnp.tanh(gi[:, 2 * D:] + r * gh[:, 2 * D:])
    o_ref[...] = (1.0 - z) * n + z * h


def _tc_gru(part, bc, h, wih_t, whh_t, bih, bhh):
    return pl.pallas_call(
        _gru_body,
        grid=(NB,),
        in_specs=[
            pl.BlockSpec((NC, BN, D), lambda nb: (0, nb, 0)),
            pl.BlockSpec((NC, BN, D), lambda nb: (0, nb, 0)),
            pl.BlockSpec((BN, D), lambda nb: (nb, 0)),
            pl.BlockSpec((D, 3 * D), lambda nb: (0, 0)),
            pl.BlockSpec((D, 3 * D), lambda nb: (0, 0)),
            pl.BlockSpec((1, 3 * D), lambda nb: (0, 0)),
            pl.BlockSpec((1, 3 * D), lambda nb: (0, 0)),
        ],
        out_specs=pl.BlockSpec((BN, D), lambda nb: (nb, 0)),
        out_shape=jax.ShapeDtypeStruct((N, D), F32),
    )(part, bc, h, wih_t, whh_t, bih, bhh)


def kernel(edge_lists, node_states, W_msg, b_msg, W_ih, W_hh, b_ih, b_hh):
    h = node_states

    gidx = (edge_lists[:, :, 0]
            + (jnp.arange(T, dtype=jnp.int32) * N)[:, None]).reshape(-1)
    tgt = edge_lists[:, :, 1].reshape(-1)
    pad = NW * C * CH - E
    gidx = jnp.concatenate([gidx, jnp.zeros((pad,), jnp.int32)])
    tgt = jnp.concatenate([tgt, jnp.full((pad,), TRASH, jnp.int32)])
    idx_all = jnp.stack(
        [gidx.reshape(NW, C, CH), tgt.reshape(NW, C, CH)], axis=2)

    zrows = jnp.zeros((ZB, D), F32)
    ones_rows = jnp.ones((CH, D), F32)

    sc_msgs, sc_binc = _sc_kernels()
    bc = sc_binc(idx_all, zrows, ones_rows)

    for layer in range(L):
        wt = W_msg[layer].reshape(T, D, D).transpose(0, 2, 1)
        bt = b_msg[layer].reshape(T, 1, D)
        prop = _tc_prop(h, wt, bt)
        part = sc_msgs(prop.reshape(T * N, D), idx_all, zrows)
        h = _tc_gru(part, bc, h,
                    W_ih[layer].T, W_hh[layer].T,
                    b_ih[layer].reshape(1, 3 * D),
                    b_hh[layer].reshape(1, 3 * D))
    return h


# speedup vs baseline: 1.6523x; 1.0066x over previous
import functools

import jax
import jax.numpy as jnp
from jax import lax
from jax.experimental import pallas as pl
from jax.experimental.pallas import tpu as pltpu
from jax.experimental.pallas import tpu_sc as plsc

N = 10000
D = 128
T = 6
L = 2
M = 100000
E = T * M
SMALL = 1e-8
F32 = jnp.float32

NC, NS = 2, 16
NW = NC * NS
CH = 128
NBUF = 3
C = NBUF * (-(-E // (NW * CH * NBUF)))
CA = C
E_PAD = NW * CA * CH
TRASH = N
SH = 10112
ZR = SH // NS
ZB = 8

BN = 1000
NB = N // BN


def _zero_shared(zrows, shared, sid):
    def zstep(i, carry):
        pltpu.sync_copy(zrows, shared.at[pl.ds(sid * ZR + i * ZB, ZB)])
        return carry
    lax.fori_loop(0, ZR // ZB, zstep, 0)


def _sc_msgs_body(prop, idx, zrows, out,
                  idxb, r0, r1, r2, g0, g1, g2, s0, s1, s2, shared):
    cid = lax.axis_index("c")
    sid = lax.axis_index("s")
    w = sid * NC + cid
    rows = (r0, r1, r2)
    sg = (g0, g1, g2)
    ss = (s0, s1, s2)
    _zero_shared(zrows, shared, sid)
    plsc.subcore_barrier()

    def step(j, carry):
        c = NBUF * j
        pltpu.sync_copy(idx.at[w, pl.ds(c, NBUF)], idxb)
        cps = [pltpu.async_copy(prop.at[idxb.at[b, 0]], rows[b], sg[b])
               for b in range(NBUF)]
        scs = []
        for b in range(NBUF):
            cps[b].wait()
            scs.append(pltpu.async_copy(rows[b], shared.at[idxb.at[b, 1]],
                                        ss[b], add=True))
        for d in scs:
            d.wait()
        return carry

    lax.fori_loop(0, C // NBUF, step, 0)
    plsc.subcore_barrier()

    @pl.when(sid < 10)
    def _():
        pltpu.sync_copy(shared.at[pl.ds(sid * 1000, 1000)],
                        out.at[cid, pl.ds(sid * 1000, 1000)])


@functools.cache
def _sc_kernels():
    mesh = plsc.VectorSubcoreMesh(core_axis_name="c", subcore_axis_name="s",
                                  num_cores=NC, num_subcores=NS)
    sc_msgs = pl.kernel(
        _sc_msgs_body,
        out_type=jax.ShapeDtypeStruct((NC, N, D), F32),
        mesh=mesh,
        scratch_types=(
            [pltpu.VMEM((NBUF, 2, CH), jnp.int32)]
            + [pltpu.VMEM((CH, D), F32)] * NBUF
            + [pltpu.SemaphoreType.DMA] * (2 * NBUF)
            + [pltpu.VMEM_SHARED((SH, D), F32)]
        ),
    )
    sc_binc = pl.kernel(
        _sc_binc_body,
        out_type=jax.ShapeDtypeStruct((NC, N, D), F32),
        mesh=mesh,
        scratch_types=[
            pltpu.VMEM((NBUF, 2, CH), jnp.int32),
            pltpu.VMEM((CH, D), F32),
            pltpu.VMEM_SHARED((SH, D), F32),
        ],
    )
    return sc_msgs, sc_binc


def _sc_binc_body(idx, zrows, ones_h, out, idxb, ones_v, shared):
    cid = lax.axis_index("c")
    sid = lax.axis_index("s")
    w = sid * NC + cid
    _zero_shared(zrows, shared, sid)
    pltpu.sync_copy(ones_h, ones_v)
    plsc.subcore_barrier()

    def step(j, carry):
        c = NBUF * j
        pltpu.sync_copy(idx.at[w, pl.ds(c, NBUF)], idxb)
        for b in range(NBUF):
            pltpu.sync_copy(ones_v, shared.at[idxb.at[b, 1]], add=True)
        return carry

    lax.fori_loop(0, C // NBUF, step, 0)
    plsc.subcore_barrier()

    @pl.when(sid < 10)
    def _():
        pltpu.sync_copy(shared.at[pl.ds(sid * 1000, 1000)],
                        out.at[cid, pl.ds(sid * 1000, 1000)])


def _prop_body(h_ref, w_ref, b_ref, o_ref):
    o_ref[0] = (jnp.dot(h_ref[...], w_ref[0], preferred_element_type=F32)
                + b_ref[0])


def _tc_prop(h, wt, bt):
    return pl.pallas_call(
        _prop_body,
        grid=(NB, T),
        in_specs=[
            pl.BlockSpec((BN, D), lambda nb, t: (nb, 0)),
            pl.BlockSpec((1, D, D), lambda nb, t: (t, 0, 0)),
            pl.BlockSpec((1, 1, D), lambda nb, t: (t, 0, 0)),
        ],
        out_specs=pl.BlockSpec((1, BN, D), lambda nb, t: (t, nb, 0)),
        out_shape=jax.ShapeDtypeStruct((T, N, D), F32),
    )(h, wt, bt)


def _gru_body(p_ref, bc_ref, h_ref, wih_ref, whh_ref, bih_ref, bhh_ref, o_ref):
    cnt = bc_ref[0, :, 0] + bc_ref[1, :, 0]
    div = jnp.where(cnt == 0.0, 1.0, cnt)
    msgs = (p_ref[0] + p_ref[1]) / div[:, None] + SMALL
    h = h_ref[...]
    gi = jnp.dot(msgs, wih_ref[...], preferred_element_type=F32) + bih_ref[...]
    gh = jnp.dot(h, whh_ref[...], preferred_element_type=F32) + bhh_ref[...]
    r = jax.nn.sigmoid(gi[:, :D] + gh[:, :D])
    z = jax.nn.sigmoid(gi[:, D:2 * D] + gh[:, D:2 * D])
    n = jnp.tanh(gi[:, 2 * D:] + r * gh[:, 2 * D:])
    o_ref[...] = (1.0 - z) * n + z * h


def _tc_gru(part, bc, h, wih_t, whh_t, bih, bhh):
    return pl.pallas_call(
        _gru_body,
        grid=(NB,),
        in_specs=[
            pl.BlockSpec((NC, BN, D), lambda nb: (0, nb, 0)),
            pl.BlockSpec((NC, BN, D), lambda nb: (0, nb, 0)),
            pl.BlockSpec((BN, D), lambda nb: (nb, 0)),
            pl.BlockSpec((D, 3 * D), lambda nb: (0, 0)),
            pl.BlockSpec((D, 3 * D), lambda nb: (0, 0)),
            pl.BlockSpec((1, 3 * D), lambda nb: (0, 0)),
            pl.BlockSpec((1, 3 * D), lambda nb: (0, 0)),
        ],
        out_specs=pl.BlockSpec((BN, D), lambda nb: (nb, 0)),
        out_shape=jax.ShapeDtypeStruct((N, D), F32),
    )(part, bc, h, wih_t, whh_t, bih, bhh)


def kernel(edge_lists, node_states, W_msg, b_msg, W_ih, W_hh, b_ih, b_hh):
    h = node_states

    gidx = (edge_lists[:, :, 0]
            + (jnp.arange(T, dtype=jnp.int32) * N)[:, None]).reshape(-1)
    tgt = edge_lists[:, :, 1].reshape(-1)
    pad = NW * C * CH - E
    gidx = jnp.concatenate([gidx, jnp.zeros((pad,), jnp.int32)])
    tgt = jnp.concatenate([tgt, jnp.full((pad,), TRASH, jnp.int32)])
    idx_all = jnp.stack(
        [gidx.reshape(NW, C, CH), tgt.reshape(NW, C, CH)], axis=2)

    zrows = jnp.zeros((ZB, D), F32)
    ones_rows = jnp.ones((CH, D), F32)

    sc_msgs, sc_binc = _sc_kernels()
    bc = sc_binc(idx_all, zrows, ones_rows)

    for layer in range(L):
        wt = W_msg[layer].reshape(T, D, D).transpose(0, 2, 1)
        bt = b_msg[layer].reshape(T, 1, D)
        prop = _tc_prop(h, wt, bt)
        part = sc_msgs(prop.reshape(T * N, D), idx_all, zrows)
        h = _tc_gru(part, bc, h,
                    W_ih[layer].T, W_hh[layer].T,
                    b_ih[layer].reshape(1, 3 * D),
                    b_hh[layer].reshape(1, 3 * D))
    return h


# speedup vs baseline: 2.0018x; 1.2115x over previous
import functools

import jax
import jax.numpy as jnp
from jax import lax
from jax.experimental import pallas as pl
from jax.experimental.pallas import tpu as pltpu
from jax.experimental.pallas import tpu_sc as plsc

N = 10000
D = 128
T = 6
L = 2
M = 100000
E = T * M
SMALL = 1e-8
F32 = jnp.float32

NC, NS = 2, 16
NW = NC * NS
CH = 128
NBUF = 3
C = NBUF * (-(-E // (NW * CH * NBUF)))
CA = C
E_PAD = NW * CA * CH
TRASH = N
SH = 10112
ZR = SH // NS
ZB = 8

BN = 1000
NB = N // BN


def _zero_shared(zrows, zbuf, shared, sid):
    for i in range(CH // ZB):
        pltpu.sync_copy(zrows, zbuf.at[pl.ds(ZB * i, ZB)])
    base = sid * ZR
    for k in range(4):
        pltpu.sync_copy(zbuf, shared.at[pl.ds(base + CH * k, CH)])
    pltpu.sync_copy(zbuf.at[pl.ds(0, ZR - 4 * CH)],
                    shared.at[pl.ds(base + 4 * CH, ZR - 4 * CH)])


def _sc_msgs_body(prop, idx, zrows, out,
                  idxb, r0, r1, r2, g0, g1, g2, s0, s1, s2, shared):
    cid = lax.axis_index("c")
    sid = lax.axis_index("s")
    w = sid * NC + cid
    rows = (r0, r1, r2)
    sg = (g0, g1, g2)
    ss = (s0, s1, s2)
    _zero_shared(zrows, r0, shared, sid)
    plsc.subcore_barrier()

    def step(j, carry):
        c = NBUF * j
        pltpu.sync_copy(idx.at[w, pl.ds(c, NBUF)], idxb)
        cps = [pltpu.async_copy(prop.at[idxb.at[b, 0]], rows[b], sg[b])
               for b in range(NBUF)]
        scs = []
        for b in range(NBUF):
            cps[b].wait()
            scs.append(pltpu.async_copy(rows[b], shared.at[idxb.at[b, 1]],
                                        ss[b], add=True))
        for d in scs:
            d.wait()
        return carry

    lax.fori_loop(0, C // NBUF, step, 0)
    plsc.subcore_barrier()

    @pl.when(sid < 10)
    def _():
        pltpu.sync_copy(shared.at[pl.ds(sid * 1000, 1000)],
                        out.at[cid, pl.ds(sid * 1000, 1000)])


@functools.cache
def _sc_kernels():
    mesh = plsc.VectorSubcoreMesh(core_axis_name="c", subcore_axis_name="s",
                                  num_cores=NC, num_subcores=NS)
    sc_msgs = pl.kernel(
        _sc_msgs_body,
        out_type=jax.ShapeDtypeStruct((NC, N, D), F32),
        mesh=mesh,
        scratch_types=(
            [pltpu.VMEM((NBUF, 2, CH), jnp.int32)]
            + [pltpu.VMEM((CH, D), F32)] * NBUF
            + [pltpu.SemaphoreType.DMA] * (2 * NBUF)
            + [pltpu.VMEM_SHARED((SH, D), F32)]
        ),
    )
    sc_binc = pl.kernel(
        _sc_binc_body,
        out_type=jax.ShapeDtypeStruct((NC, N, D), F32),
        mesh=mesh,
        scratch_types=[
            pltpu.VMEM((NBUF, 2, CH), jnp.int32),
            pltpu.VMEM((CH, D), F32),
            pltpu.VMEM_SHARED((SH, D), F32),
        ],
    )
    return sc_msgs, sc_binc


def _sc_binc_body(idx, zrows, ones_h, out, idxb, ones_v, shared):
    cid = lax.axis_index("c")
    sid = lax.axis_index("s")
    w = sid * NC + cid
    _zero_shared(zrows, ones_v, shared, sid)
    pltpu.sync_copy(ones_h, ones_v)
    plsc.subcore_barrier()

    def step(j, carry):
        c = NBUF * j
        pltpu.sync_copy(idx.at[w, pl.ds(c, NBUF)], idxb)
        for b in range(NBUF):
            pltpu.sync_copy(ones_v, shared.at[idxb.at[b, 1]], add=True)
        return carry

    lax.fori_loop(0, C // NBUF, step, 0)
    plsc.subcore_barrier()

    @pl.when(sid < 10)
    def _():
        pltpu.sync_copy(shared.at[pl.ds(sid * 1000, 1000)],
                        out.at[cid, pl.ds(sid * 1000, 1000)])


def _prop_body(h_ref, w_ref, b_ref, o_ref):
    o_ref[0] = (jnp.dot(h_ref[...], w_ref[0], preferred_element_type=F32)
                + b_ref[0])


def _tc_prop(h, wt, bt):
    return pl.pallas_call(
        _prop_body,
        grid=(NB, T),
        in_specs=[
            pl.BlockSpec((BN, D), lambda nb, t: (nb, 0)),
            pl.BlockSpec((1, D, D), lambda nb, t: (t, 0, 0)),
            pl.BlockSpec((1, 1, D), lambda nb, t: (t, 0, 0)),
        ],
        out_specs=pl.BlockSpec((1, BN, D), lambda nb, t: (t, nb, 0)),
        out_shape=jax.ShapeDtypeStruct((T, N, D), F32),
    )(h, wt, bt)


def _gru_body(p_ref, bc_ref, h_ref, wih_ref, whh_ref, bih_ref, bhh_ref, o_ref):
    cnt = bc_ref[0, :, 0] + bc_ref[1, :, 0]
    div = jnp.where(cnt == 0.0, 1.0, cnt)
    msgs = (p_ref[0] + p_ref[1]) / div[:, None] + SMALL
    h = h_ref[...]
    gi = jnp.dot(msgs, wih_ref[...], preferred_element_type=F32) + bih_ref[...]
    gh = jnp.dot(h, whh_ref[...], preferred_element_type=F32) + bhh_ref[...]
    r = jax.nn.sigmoid(gi[:, :D] + gh[:, :D])
    z = jax.nn.sigmoid(gi[:, D:2 * D] + gh[:, D:2 * D])
    n = jnp.tanh(gi[:, 2 * D:] + r * gh[:, 2 * D:])
    o_ref[...] = (1.0 - z) * n + z * h


def _tc_gru(part, bc, h, wih_t, whh_t, bih, bhh):
    return pl.pallas_call(
        _gru_body,
        grid=(NB,),
        in_specs=[
            pl.BlockSpec((NC, BN, D), lambda nb: (0, nb, 0)),
            pl.BlockSpec((NC, BN, D), lambda nb: (0, nb, 0)),
            pl.BlockSpec((BN, D), lambda nb: (nb, 0)),
            pl.BlockSpec((D, 3 * D), lambda nb: (0, 0)),
            pl.BlockSpec((D, 3 * D), lambda nb: (0, 0)),
            pl.BlockSpec((1, 3 * D), lambda nb: (0, 0)),
            pl.BlockSpec((1, 3 * D), lambda nb: (0, 0)),
        ],
        out_specs=pl.BlockSpec((BN, D), lambda nb: (nb, 0)),
        out_shape=jax.ShapeDtypeStruct((N, D), F32),
    )(part, bc, h, wih_t, whh_t, bih, bhh)


def kernel(edge_lists, node_states, W_msg, b_msg, W_ih, W_hh, b_ih, b_hh):
    h = node_states

    gidx = (edge_lists[:, :, 0]
            + (jnp.arange(T, dtype=jnp.int32) * N)[:, None]).reshape(-1)
    tgt = edge_lists[:, :, 1].reshape(-1)
    pad = NW * C * CH - E
    gidx = jnp.concatenate([gidx, jnp.zeros((pad,), jnp.int32)])
    tgt = jnp.concatenate([tgt, jnp.full((pad,), TRASH, jnp.int32)])
    idx_all = jnp.stack(
        [gidx.reshape(NW, C, CH), tgt.reshape(NW, C, CH)], axis=2)

    zrows = jnp.zeros((ZB, D), F32)
    ones_rows = jnp.ones((CH, D), F32)

    sc_msgs, sc_binc = _sc_kernels()
    bc = sc_binc(idx_all, zrows, ones_rows)

    for layer in range(L):
        wt = W_msg[layer].reshape(T, D, D).transpose(0, 2, 1)
        bt = b_msg[layer].reshape(T, 1, D)
        prop = _tc_prop(h, wt, bt)
        part = sc_msgs(prop.reshape(T * N, D), idx_all, zrows)
        h = _tc_gru(part, bc, h,
                    W_ih[layer].T, W_hh[layer].T,
                    b_ih[layer].reshape(1, 3 * D),
                    b_hh[layer].reshape(1, 3 * D))
    return h
